# baseline (speedup 1.0000x reference)
import functools

import jax
import jax.numpy as jnp
from jax import lax
from jax.experimental import pallas as pl
from jax.experimental.pallas import tpu as pltpu, tpu_sc as plsc

NC = 2
NS = 16
LW = 16
K_CH = 80
R_BLK = 1000
NIDX = 8
NROW = 4


def _sc_mesh():
    return plsc.VectorSubcoreMesh(core_axis_name="c", subcore_axis_name="s")


def _sc_degree(dst, zeros_nl, ones_kl):
    npad = zeros_nl.shape[0]
    e = dst.shape[0]
    ec = e // NC
    et = ec // NS
    nch = et // K_CH
    rt = npad // NS

    @functools.partial(
        pl.kernel,
        out_type=jax.ShapeDtypeStruct((NC, npad, LW), jnp.float32),
        mesh=_sc_mesh(),
        scratch_types=[
            pltpu.VMEM_SHARED((npad, LW), jnp.float32),
            pltpu.VMEM((K_CH,), jnp.int32),
            pltpu.VMEM((K_CH, LW), jnp.float32),
        ],
    )
    def k(dst_hbm, zeros_hbm, ones_hbm, out_hbm, acc, didx, ones_v):
        c = lax.axis_index("c")
        s = lax.axis_index("s")
        rbase = s * rt
        pltpu.sync_copy(zeros_hbm.at[pl.ds(rbase, rt)], acc.at[pl.ds(rbase, rt)])
        pltpu.sync_copy(ones_hbm, ones_v)
        plsc.subcore_barrier()
        ebase = c * ec + s * et

        def body(j, carry):
            off = pl.multiple_of(ebase + j * K_CH, 8)
            pltpu.sync_copy(dst_hbm.at[pl.ds(off, K_CH)], didx)
            pltpu.sync_copy(ones_v, acc.at[didx], add=True)
            return carry

        lax.fori_loop(0, nch, body, 0)
        plsc.subcore_barrier()
        pltpu.sync_copy(acc.at[pl.ds(rbase, rt)], out_hbm.at[c, pl.ds(rbase, rt)])

    return k(dst, zeros_nl, ones_kl)


def _sc_agg(table, src, dst, zeros_nd):
    n, d = table.shape
    npad = zeros_nd.shape[0]
    e = src.shape[0]
    ec = e // NC
    et = ec // NS
    nch = et // K_CH
    rt = npad // NS

    pipelined = nch >= 2

    @functools.partial(
        pl.kernel,
        out_type=jax.ShapeDtypeStruct((NC, npad, d), jnp.float32),
        mesh=_sc_mesh(),
        scratch_types=[
            pltpu.VMEM_SHARED((npad, d), jnp.float32),
            pltpu.VMEM((K_CH,), jnp.int32),
            pltpu.VMEM((K_CH,), jnp.int32),
            pltpu.VMEM((K_CH,), jnp.int32),
            pltpu.VMEM((K_CH,), jnp.int32),
            pltpu.VMEM((K_CH, d), jnp.float32),
            pltpu.VMEM((K_CH, d), jnp.float32),
            pltpu.SemaphoreType.DMA,
            pltpu.SemaphoreType.DMA,
        ],
    )
    def k(table_hbm, src_hbm, dst_hbm, zeros_hbm, out_hbm, acc,
          sidx0, sidx1, didx0, didx1, rows0, rows1, sem0, sem1):
        c = lax.axis_index("c")
        s = lax.axis_index("s")
        rbase = s * rt
        pltpu.sync_copy(zeros_hbm.at[pl.ds(rbase, rt)], acc.at[pl.ds(rbase, rt)])
        plsc.subcore_barrier()
        ebase = c * ec + s * et

        def fetch(j, sidx, didx):
            off = pl.multiple_of(ebase + j * K_CH, 8)
            pltpu.sync_copy(src_hbm.at[pl.ds(off, K_CH)], sidx)
            pltpu.sync_copy(dst_hbm.at[pl.ds(off, K_CH)], didx)

        if pipelined:
            def body(jj, carry):
                j = jj * 2
                fetch(j, sidx0, didx0)
                cp0 = pltpu.async_copy(table_hbm.at[sidx0], rows0, sem0)
                fetch(j + 1, sidx1, didx1)
                cp1 = pltpu.async_copy(table_hbm.at[sidx1], rows1, sem1)
                cp0.wait()
                pltpu.sync_copy(rows0, acc.at[didx0], add=True)
                cp1.wait()
                pltpu.sync_copy(rows1, acc.at[didx1], add=True)
                return carry

            lax.fori_loop(0, nch // 2, body, 0)
            if nch % 2 == 1:
                fetch(nch - 1, sidx0, didx0)
                pltpu.async_copy(table_hbm.at[sidx0], rows0, sem0).wait()
                pltpu.sync_copy(rows0, acc.at[didx0], add=True)
        else:
            def body(j, carry):
                fetch(j, sidx0, didx0)
                pltpu.async_copy(table_hbm.at[sidx0], rows0, sem0).wait()
                pltpu.sync_copy(rows0, acc.at[didx0], add=True)
                return carry

            lax.fori_loop(0, nch, body, 0)

        plsc.subcore_barrier()
        pltpu.sync_copy(acc.at[pl.ds(rbase, rt)], out_hbm.at[c, pl.ds(rbase, rt)])

    return k(table, src, dst, zeros_nd)


def _deg_terms(dp_ref):
    deg = dp_ref[0, :, 0:1] + dp_ref[1, :, 0:1] + 1.0
    return lax.rsqrt(deg), 1.0 / deg


def _tc_k1(x, w1, degparts):
    n, d = x.shape
    g = n // R_BLK

    def body(x_ref, w_ref, dp_ref, h_ref, hs_ref):
        dis, _ = _deg_terms(dp_ref)
        h = jnp.dot(x_ref[...], w_ref[...], preferred_element_type=jnp.float32)
        h_ref[...] = h
        hs_ref[...] = h * dis

    return pl.pallas_call(
        body,
        grid=(g,),
        in_specs=[
            pl.BlockSpec((R_BLK, d), lambda i: (i, 0)),
            pl.BlockSpec((d, d), lambda i: (0, 0)),
            pl.BlockSpec((NC, R_BLK, LW), lambda i: (0, i, 0)),
        ],
        out_specs=[
            pl.BlockSpec((R_BLK, d), lambda i: (i, 0)),
            pl.BlockSpec((R_BLK, d), lambda i: (i, 0)),
        ],
        out_shape=[
            jax.ShapeDtypeStruct((n, d), jnp.float32),
            jax.ShapeDtypeStruct((n, d), jnp.float32),
        ],
    )(x, w1, degparts)


def _tc_k2(sparts, h, degparts, b, w2):
    n, d = h.shape
    g = n // R_BLK

    def body(sp_ref, h_ref, dp_ref, b_ref, w_ref, h2_ref, hs2_ref):
        dis, inv = _deg_terms(dp_ref)
        s = sp_ref[0] + sp_ref[1]
        z = jax.nn.gelu(s * dis + h_ref[...] * inv + b_ref[...])
        h2 = jnp.dot(z, w_ref[...], preferred_element_type=jnp.float32)
        h2_ref[...] = h2
        hs2_ref[...] = h2 * dis

    return pl.pallas_call(
        body,
        grid=(g,),
        in_specs=[
            pl.BlockSpec((NC, R_BLK, d), lambda i: (0, i, 0)),
            pl.BlockSpec((R_BLK, d), lambda i: (i, 0)),
            pl.BlockSpec((NC, R_BLK, LW), lambda i: (0, i, 0)),
            pl.BlockSpec((1, d), lambda i: (0, 0)),
            pl.BlockSpec((d, d), lambda i: (0, 0)),
        ],
        out_specs=[
            pl.BlockSpec((R_BLK, d), lambda i: (i, 0)),
            pl.BlockSpec((R_BLK, d), lambda i: (i, 0)),
        ],
        out_shape=[
            jax.ShapeDtypeStruct((n, d), jnp.float32),
            jax.ShapeDtypeStruct((n, d), jnp.float32),
        ],
    )(sparts, h, degparts, b, w2)


def _tc_k3(sparts, h, degparts, b, batch2d, wc, bc):
    n, d = h.shape
    g = n // R_BLK
    b_seg = 64

    def body(sp_ref, h_ref, dp_ref, b_ref, bt_ref, wc_ref, bc_ref, out_ref,
             sums, counts):
        i = pl.program_id(0)

        @pl.when(i == 0)
        def _():
            sums[...] = jnp.zeros_like(sums)
            counts[...] = jnp.zeros_like(counts)

        dis, inv = _deg_terms(dp_ref)
        s = sp_ref[0] + sp_ref[1]
        z = jax.nn.gelu(s * dis + h_ref[...] * inv + b_ref[...])
        oh = (bt_ref[...] == lax.broadcasted_iota(jnp.int32, (R_BLK, b_seg), 1)
              ).astype(jnp.float32)
        sums[...] += lax.dot_general(oh, z, (((0,), (0,)), ((), ())),
                                     preferred_element_type=jnp.float32)
        counts[...] += lax.dot_general(oh, jnp.ones_like(z),
                                       (((0,), (0,)), ((), ())),
                                       preferred_element_type=jnp.float32)

        @pl.when(i == g - 1)
        def _():
            gm = sums[...] / jnp.maximum(counts[...], 1.0)
            out_ref[...] = jnp.dot(gm, wc_ref[...],
                                   preferred_element_type=jnp.float32) + bc_ref[...]

    return pl.pallas_call(
        body,
        grid=(g,),
        in_specs=[
            pl.BlockSpec((NC, R_BLK, d), lambda i: (0, i, 0)),
            pl.BlockSpec((R_BLK, d), lambda i: (i, 0)),
            pl.BlockSpec((NC, R_BLK, LW), lambda i: (0, i, 0)),
            pl.BlockSpec((1, d), lambda i: (0, 0)),
            pl.BlockSpec((R_BLK, 1), lambda i: (i, 0)),
            pl.BlockSpec((d, wc.shape[1]), lambda i: (0, 0)),
            pl.BlockSpec((1, wc.shape[1]), lambda i: (0, 0)),
        ],
        out_specs=pl.BlockSpec((b_seg, wc.shape[1]), lambda i: (0, 0)),
        out_shape=jax.ShapeDtypeStruct((b_seg, wc.shape[1]), jnp.float32),
        scratch_shapes=[
            pltpu.VMEM((b_seg, d), jnp.float32),
            pltpu.VMEM((b_seg, d), jnp.float32),
        ],
    )(sparts, h, degparts, b, batch2d, wc, bc)


def kernel(x, edge_index, batch, W1, b1, W2, b2, Wc, bc):
    n, d = x.shape
    src = edge_index[0]
    dst = edge_index[1]
    npad = -(-n // (NS * 8)) * (NS * 8)
    zeros_nl = jnp.zeros((npad, LW), jnp.float32)
    ones_kl = jnp.ones((K_CH, LW), jnp.float32)
    zeros_nd = jnp.zeros((npad, d), jnp.float32)

    degparts = _sc_degree(dst, zeros_nl, ones_kl)
    h1, hs1 = _tc_k1(x, W1, degparts)
    s1 = _sc_agg(hs1, src, dst, zeros_nd)
    h2, hs2 = _tc_k2(s1, h1, degparts, b1.reshape(1, d), W2)
    s2 = _sc_agg(hs2, src, dst, zeros_nd)
    out = _tc_k3(s2, h2, degparts, b2.reshape(1, d),
                 batch.reshape(n, 1), Wc, bc.reshape(1, -1))
    return out
